# Initial kernel scaffold; baseline (speedup 1.0000x reference)
import jax
import jax.numpy as jnp
from jax.experimental import pallas as pl


def kernel(x, edge_index, edge_attr, batch, Wn, bn_b, We, be, Wf, bf, Ws, bs, g1, b1, Wh, bh, gh, bh2, Wo, bo):
    raise NotImplementedError("write your pallas kernel here")


# speedup vs baseline: 2.7279x; 2.7279x over previous
import functools

import jax
import jax.numpy as jnp
from jax import lax
from jax.experimental import pallas as pl
from jax.experimental.pallas import tpu as pltpu
from jax.experimental.pallas import tpu_sc as plsc

N = 10000
E = 320000
D = 128
DE = 16
H = 128
G = 64
L = 3

NC = 2
NS = 16
NW = NC * NS
EPW = E // NW
KCH = 80
NCHUNK = EPW // KCH
NP = 10240
ROWS_PER_TILE = NP // NS

_mesh = plsc.VectorSubcoreMesh(core_axis_name="c", subcore_axis_name="s")


def _sc_gather_body(h_hbm, dst_hbm, src_hbm, hd_hbm, hs_hbm,
                    didx, sidx, dbuf, sbuf, sem_d, sem_s):
    wid = lax.axis_index("s") * NC + lax.axis_index("c")
    base = wid * EPW

    def chunk(i, carry):
        off = base + i * KCH
        pltpu.sync_copy(dst_hbm.at[pl.ds(off, KCH)], didx)
        pltpu.sync_copy(src_hbm.at[pl.ds(off, KCH)], sidx)
        cd = pltpu.async_copy(h_hbm.at[didx], dbuf, sem_d)
        cs = pltpu.async_copy(h_hbm.at[sidx], sbuf, sem_s)
        cd.wait()
        cs.wait()
        pltpu.sync_copy(dbuf, hd_hbm.at[pl.ds(off, KCH)])
        pltpu.sync_copy(sbuf, hs_hbm.at[pl.ds(off, KCH)])
        return carry

    lax.fori_loop(0, NCHUNK, chunk, 0)


_sc_gather = pl.kernel(
    _sc_gather_body,
    out_type=(jax.ShapeDtypeStruct((E, H), jnp.float32),
              jax.ShapeDtypeStruct((E, H), jnp.float32)),
    mesh=_mesh,
    scratch_types=[
        pltpu.VMEM((KCH,), jnp.int32),
        pltpu.VMEM((KCH,), jnp.int32),
        pltpu.VMEM((KCH, H), jnp.float32),
        pltpu.VMEM((KCH, H), jnp.float32),
        pltpu.SemaphoreType.DMA,
        pltpu.SemaphoreType.DMA,
    ],
)


def _sc_scatter_body(msg_hbm, dst_hbm, zeros_hbm, out_hbm, didx, mbuf, acc):
    cid = lax.axis_index("c")
    sid = lax.axis_index("s")
    wid = sid * NC + cid
    base = wid * EPW

    pltpu.sync_copy(zeros_hbm.at[pl.ds(sid * ROWS_PER_TILE, ROWS_PER_TILE)],
                    acc.at[pl.ds(sid * ROWS_PER_TILE, ROWS_PER_TILE)])
    plsc.subcore_barrier()

    def chunk(i, carry):
        off = base + i * KCH
        pltpu.sync_copy(dst_hbm.at[pl.ds(off, KCH)], didx)
        pltpu.sync_copy(msg_hbm.at[pl.ds(off, KCH)], mbuf)
        pltpu.sync_copy(mbuf, acc.at[didx], add=True)
        return carry

    lax.fori_loop(0, NCHUNK, chunk, 0)
    plsc.subcore_barrier()
    pltpu.sync_copy(acc.at[pl.ds(sid * ROWS_PER_TILE, ROWS_PER_TILE)],
                    out_hbm.at[pl.ds(cid * NP + sid * ROWS_PER_TILE,
                                     ROWS_PER_TILE)])


_sc_scatter = pl.kernel(
    _sc_scatter_body,
    out_type=jax.ShapeDtypeStruct((2 * NP, H), jnp.float32),
    mesh=_mesh,
    scratch_types=[
        pltpu.VMEM((KCH,), jnp.int32),
        pltpu.VMEM((KCH, H), jnp.float32),
        pltpu.VMEM_SHARED((NP, H), jnp.float32),
    ],
)


def _sigmoid(v):
    return jax.nn.sigmoid(v)


def _embed_body(x_ref, w_ref, b_ref, o_ref):
    o_ref[...] = (jnp.dot(x_ref[...], w_ref[...],
                          preferred_element_type=jnp.float32) + b_ref[...])


_embed = pl.pallas_call(
    _embed_body,
    out_shape=jax.ShapeDtypeStruct((N, H), jnp.float32),
)

BE = 4000


def _eembed_body(ea_ref, w_ref, b_ref, o_ref):
    o_ref[...] = (jnp.dot(ea_ref[...], w_ref[...],
                          preferred_element_type=jnp.float32) + b_ref[...])


_eembed = pl.pallas_call(
    _eembed_body,
    grid=(E // BE,),
    in_specs=[
        pl.BlockSpec((BE, DE), lambda i: (i, 0)),
        pl.BlockSpec((DE, H), lambda i: (0, 0)),
        pl.BlockSpec((1, H), lambda i: (0, 0)),
    ],
    out_specs=pl.BlockSpec((BE, H), lambda i: (i, 0)),
    out_shape=jax.ShapeDtypeStruct((E, H), jnp.float32),
)


def _msg_body(hd_ref, hs_ref, e_ref, w_ref, b_ref, o_ref):
    z = jnp.concatenate([hd_ref[...], hs_ref[...], e_ref[...]], axis=1)
    logits = (jnp.dot(z, w_ref[...], preferred_element_type=jnp.float32)
              + b_ref[...])
    gl = logits[:, :H]
    cl = logits[:, H:]
    o_ref[...] = jax.nn.sigmoid(gl) * jax.nn.softplus(cl)


_msg = pl.pallas_call(
    _msg_body,
    grid=(E // BE,),
    in_specs=[
        pl.BlockSpec((BE, H), lambda i: (i, 0)),
        pl.BlockSpec((BE, H), lambda i: (i, 0)),
        pl.BlockSpec((BE, H), lambda i: (i, 0)),
        pl.BlockSpec((3 * H, 2 * H), lambda i: (0, 0)),
        pl.BlockSpec((1, 2 * H), lambda i: (0, 0)),
    ],
    out_specs=pl.BlockSpec((BE, H), lambda i: (i, 0)),
    out_shape=jax.ShapeDtypeStruct((E, H), jnp.float32),
)


def _update_body(p_ref, h_ref, g_ref, b_ref, o_ref):
    agg = p_ref[:N, :] + p_ref[NP:NP + N, :]
    mu = jnp.mean(agg, axis=0, keepdims=True)
    xc = agg - mu
    var = jnp.mean(xc * xc, axis=0, keepdims=True)
    xn = xc / jnp.sqrt(var + 1e-5) * g_ref[...] + b_ref[...]
    o_ref[...] = h_ref[...] + xn * jax.nn.sigmoid(xn)


_update = pl.pallas_call(
    _update_body,
    out_shape=jax.ShapeDtypeStruct((N, H), jnp.float32),
)


def _head_body(h_ref, batch_ref, wh_ref, bh_ref, gh_ref, bh2_ref, wo_ref,
               bo_ref, o_ref):
    ids = lax.broadcasted_iota(jnp.int32, (G, N), 0)
    onehot = jnp.where(ids == batch_ref[...], 1.0, 0.0).astype(jnp.float32)
    pooled = jnp.dot(onehot, h_ref[...], preferred_element_type=jnp.float32,
                     precision=lax.Precision.HIGHEST)
    t0 = (jnp.dot(pooled, wh_ref[...], preferred_element_type=jnp.float32)
          + bh_ref[...])
    mu = jnp.mean(t0, axis=0, keepdims=True)
    xc = t0 - mu
    var = jnp.mean(xc * xc, axis=0, keepdims=True)
    xn = xc / jnp.sqrt(var + 1e-5) * gh_ref[...] + bh2_ref[...]
    t = xn * jax.nn.sigmoid(xn)
    o_ref[...] = (jnp.dot(t, wo_ref[...], preferred_element_type=jnp.float32)
                  + bo_ref[...])


_head = pl.pallas_call(
    _head_body,
    out_shape=jax.ShapeDtypeStruct((G, 1), jnp.float32),
)


def kernel(x, edge_index, edge_attr, batch, Wn, bn_b, We, be, Wf, bf, Ws, bs,
           g1, b1, Wh, bh, gh, bh2, Wo, bo):
    src = edge_index[0]
    dst = edge_index[1]
    h = _embed(x, Wn, bn_b.reshape(1, H))
    e = _eembed(edge_attr, We, be.reshape(1, H))
    zeros = jnp.zeros((NP, H), jnp.float32)
    for l in range(L):
        w = jnp.concatenate([Wf[l], Ws[l]], axis=1)
        b2 = jnp.concatenate([bf[l], bs[l]]).reshape(1, 2 * H)
        hd, hs = _sc_gather(h, dst, src)
        msg = _msg(hd, hs, e, w, b2)
        parts = _sc_scatter(msg, dst, zeros)
        h = _update(parts, h, g1[l].reshape(1, H), b1[l].reshape(1, H))
    return _head(h, batch.reshape(1, N), Wh, bh.reshape(1, H),
                 gh.reshape(1, H), bh2.reshape(1, H), Wo, bo.reshape(1, 1))


# speedup vs baseline: 3.5600x; 1.3050x over previous
import functools

import jax
import jax.numpy as jnp
from jax import lax
from jax.experimental import pallas as pl
from jax.experimental.pallas import tpu as pltpu
from jax.experimental.pallas import tpu_sc as plsc

N = 10000
E = 320000
D = 128
DE = 16
H = 128
G = 64
L = 3

HP = H // 2
EH = E // 2

NC = 2
NS = 16
NW = NC * NS
EPW = E // NW
KCH = 80
NCHUNK = EPW // KCH
GB = 400
GSUB = 80
NGB = EPW // GB
NSUB = GB // GSUB
NP = 10240
ROWS_PER_TILE = NP // NS

_mesh = plsc.VectorSubcoreMesh(core_axis_name="c", subcore_axis_name="s")


def _sc_gather_body(hpk_hbm, dst_hbm, src_hbm, hd_hbm, hs_hbm,
                    didx, sidx, dbuf, sbuf, sem_d, sem_s):
    wid = lax.axis_index("s") * NC + lax.axis_index("c")
    base = wid * EPW

    def make_blk(lane, rbase):
        def blk(i, carry):
            off = base + i * GB
            row = rbase + i * GB
            pltpu.sync_copy(dst_hbm.at[pl.ds(off, GB)], didx)
            pltpu.sync_copy(src_hbm.at[pl.ds(off, GB)], sidx)
            copies = []
            for j in range(NSUB):
                sl = pl.ds(j * GSUB, GSUB)
                copies.append(pltpu.async_copy(hpk_hbm.at[didx.at[sl]],
                                               dbuf.at[sl], sem_d))
                copies.append(pltpu.async_copy(hpk_hbm.at[sidx.at[sl]],
                                               sbuf.at[sl], sem_s))
            for c in copies:
                c.wait()
            pltpu.sync_copy(dbuf, hd_hbm.at[pl.ds(row, GB), pl.ds(lane, HP)])
            pltpu.sync_copy(sbuf, hs_hbm.at[pl.ds(row, GB), pl.ds(lane, HP)])
            return carry
        return blk

    @pl.when(wid < NS)
    def _():
        lax.fori_loop(0, NGB, make_blk(0, wid * EPW), 0)

    @pl.when(wid >= NS)
    def _():
        lax.fori_loop(0, NGB, make_blk(HP, (wid - NS) * EPW), 0)


_sc_gather = pl.kernel(
    _sc_gather_body,
    out_type=(jax.ShapeDtypeStruct((EH, H), jnp.uint32),
              jax.ShapeDtypeStruct((EH, H), jnp.uint32)),
    mesh=_mesh,
    compiler_params=pltpu.CompilerParams(use_tc_tiling_on_sc=False),
    scratch_types=[
        pltpu.VMEM((GB,), jnp.int32),
        pltpu.VMEM((GB,), jnp.int32),
        pltpu.VMEM((GB, HP), jnp.uint32),
        pltpu.VMEM((GB, HP), jnp.uint32),
        pltpu.SemaphoreType.DMA,
        pltpu.SemaphoreType.DMA,
    ],
)


def _sc_scatter_body(msgA_hbm, msgB_hbm, dst_hbm, zeros_hbm, out_hbm,
                     didx, mbuf, acc):
    cid = lax.axis_index("c")
    sid = lax.axis_index("s")
    wid = sid * NC + cid
    base = wid * EPW

    pltpu.sync_copy(zeros_hbm.at[pl.ds(sid * ROWS_PER_TILE, ROWS_PER_TILE)],
                    acc.at[pl.ds(sid * ROWS_PER_TILE, ROWS_PER_TILE)])
    plsc.subcore_barrier()

    def make_chunk(msg_hbm, rbase):
        def chunk(i, carry):
            off = base + i * KCH
            row = rbase + i * KCH
            pltpu.sync_copy(dst_hbm.at[pl.ds(off, KCH)], didx)
            pltpu.sync_copy(msg_hbm.at[pl.ds(row, KCH)], mbuf)
            pltpu.sync_copy(mbuf, acc.at[didx], add=True)
            return carry
        return chunk

    @pl.when(wid < NS)
    def _():
        lax.fori_loop(0, NCHUNK, make_chunk(msgA_hbm, wid * EPW), 0)

    @pl.when(wid >= NS)
    def _():
        lax.fori_loop(0, NCHUNK, make_chunk(msgB_hbm, (wid - NS) * EPW), 0)

    plsc.subcore_barrier()
    pltpu.sync_copy(acc.at[pl.ds(sid * ROWS_PER_TILE, ROWS_PER_TILE)],
                    out_hbm.at[pl.ds(cid * NP + sid * ROWS_PER_TILE,
                                     ROWS_PER_TILE)])


_sc_scatter = pl.kernel(
    _sc_scatter_body,
    out_type=jax.ShapeDtypeStruct((2 * NP, H), jnp.float32),
    mesh=_mesh,
    scratch_types=[
        pltpu.VMEM((KCH,), jnp.int32),
        pltpu.VMEM((KCH, H), jnp.float32),
        pltpu.VMEM_SHARED((NP, H), jnp.float32),
    ],
)


def _pack(x):
    u = lax.bitcast_convert_type(x, jnp.uint32)
    r = (u + jnp.uint32(0x7FFF) + ((u >> 16) & jnp.uint32(1))) >> 16
    return r[:, :HP] | (r[:, HP:] << 16)


def _unpack(pk):
    lo = lax.bitcast_convert_type(pk << 16, jnp.float32)
    hi = lax.bitcast_convert_type(pk & jnp.uint32(0xFFFF0000), jnp.float32)
    return jnp.concatenate([lo, hi], axis=1)


def _embed_body(x_ref, w_ref, b_ref, o_ref, opk_ref):
    h0 = (jnp.dot(x_ref[...], w_ref[...],
                  preferred_element_type=jnp.float32) + b_ref[...])
    o_ref[...] = h0
    opk_ref[...] = _pack(h0)


_embed = pl.pallas_call(
    _embed_body,
    out_shape=(jax.ShapeDtypeStruct((N, H), jnp.float32),
               jax.ShapeDtypeStruct((N, HP), jnp.uint32)),
)

BEH = 2000


def _eembed_body(eaA_ref, eaB_ref, w_ref, b_ref, o_ref):
    eA = (jnp.dot(eaA_ref[...], w_ref[...],
                  preferred_element_type=jnp.float32) + b_ref[...])
    eB = (jnp.dot(eaB_ref[...], w_ref[...],
                  preferred_element_type=jnp.float32) + b_ref[...])
    o_ref[...] = jnp.concatenate([_pack(eA), _pack(eB)], axis=1)


_eembed = pl.pallas_call(
    _eembed_body,
    grid=(EH // BEH,),
    in_specs=[
        pl.BlockSpec((BEH, DE), lambda i: (i, 0)),
        pl.BlockSpec((BEH, DE), lambda i: (i + EH // BEH, 0)),
        pl.BlockSpec((DE, H), lambda i: (0, 0)),
        pl.BlockSpec((1, H), lambda i: (0, 0)),
    ],
    out_specs=pl.BlockSpec((BEH, H), lambda i: (i, 0)),
    out_shape=jax.ShapeDtypeStruct((EH, H), jnp.uint32),
)


def _msg_body(hd_ref, hs_ref, e_ref, w_ref, b_ref, oA_ref, oB_ref):
    hdp = hd_ref[...]
    hsp = hs_ref[...]
    ep = e_ref[...]
    zA = jnp.concatenate([_unpack(hdp[:, :HP]), _unpack(hsp[:, :HP]),
                          _unpack(ep[:, :HP])], axis=1)
    zB = jnp.concatenate([_unpack(hdp[:, HP:]), _unpack(hsp[:, HP:]),
                          _unpack(ep[:, HP:])], axis=1)
    z = jnp.concatenate([zA, zB], axis=0)
    logits = (jnp.dot(z, w_ref[...], preferred_element_type=jnp.float32)
              + b_ref[...])
    msg = jax.nn.sigmoid(logits[:, :H]) * jax.nn.softplus(logits[:, H:])
    oA_ref[...] = msg[:BEH]
    oB_ref[...] = msg[BEH:]


_msg = pl.pallas_call(
    _msg_body,
    grid=(EH // BEH,),
    in_specs=[
        pl.BlockSpec((BEH, H), lambda i: (i, 0)),
        pl.BlockSpec((BEH, H), lambda i: (i, 0)),
        pl.BlockSpec((BEH, H), lambda i: (i, 0)),
        pl.BlockSpec((3 * H, 2 * H), lambda i: (0, 0)),
        pl.BlockSpec((1, 2 * H), lambda i: (0, 0)),
    ],
    out_specs=(pl.BlockSpec((BEH, H), lambda i: (i, 0)),
               pl.BlockSpec((BEH, H), lambda i: (i, 0))),
    out_shape=(jax.ShapeDtypeStruct((EH, H), jnp.float32),
               jax.ShapeDtypeStruct((EH, H), jnp.float32)),
)


def _update_body(p_ref, h_ref, g_ref, b_ref, o_ref, opk_ref):
    agg = p_ref[:N, :] + p_ref[NP:NP + N, :]
    mu = jnp.mean(agg, axis=0, keepdims=True)
    xc = agg - mu
    var = jnp.mean(xc * xc, axis=0, keepdims=True)
    xn = xc / jnp.sqrt(var + 1e-5) * g_ref[...] + b_ref[...]
    h_new = h_ref[...] + xn * jax.nn.sigmoid(xn)
    o_ref[...] = h_new
    opk_ref[...] = _pack(h_new)


_update = pl.pallas_call(
    _update_body,
    out_shape=(jax.ShapeDtypeStruct((N, H), jnp.float32),
               jax.ShapeDtypeStruct((N, HP), jnp.uint32)),
)


def _head_body(h_ref, batch_ref, wh_ref, bh_ref, gh_ref, bh2_ref, wo_ref,
               bo_ref, o_ref):
    ids = lax.broadcasted_iota(jnp.int32, (G, N), 0)
    onehot = jnp.where(ids == batch_ref[...], 1.0, 0.0).astype(jnp.float32)
    pooled = jnp.dot(onehot, h_ref[...], preferred_element_type=jnp.float32,
                     precision=lax.Precision.HIGHEST)
    t0 = (jnp.dot(pooled, wh_ref[...], preferred_element_type=jnp.float32)
          + bh_ref[...])
    mu = jnp.mean(t0, axis=0, keepdims=True)
    xc = t0 - mu
    var = jnp.mean(xc * xc, axis=0, keepdims=True)
    xn = xc / jnp.sqrt(var + 1e-5) * gh_ref[...] + bh2_ref[...]
    t = xn * jax.nn.sigmoid(xn)
    o_ref[...] = (jnp.dot(t, wo_ref[...], preferred_element_type=jnp.float32)
                  + bo_ref[...])


_head = pl.pallas_call(
    _head_body,
    out_shape=jax.ShapeDtypeStruct((G, 1), jnp.float32),
)


def kernel(x, edge_index, edge_attr, batch, Wn, bn_b, We, be, Wf, bf, Ws, bs,
           g1, b1, Wh, bh, gh, bh2, Wo, bo):
    src = edge_index[0]
    dst = edge_index[1]
    h, hpk = _embed(x, Wn, bn_b.reshape(1, H))
    epk = _eembed(edge_attr, edge_attr, We, be.reshape(1, H))
    zeros = jnp.zeros((NP, H), jnp.float32)
    for l in range(L):
        w = jnp.concatenate([Wf[l], Ws[l]], axis=1)
        b2 = jnp.concatenate([bf[l], bs[l]]).reshape(1, 2 * H)
        hd2, hs2 = _sc_gather(hpk, dst, src)
        msgA, msgB = _msg(hd2, hs2, epk, w, b2)
        parts = _sc_scatter(msgA, msgB, dst, zeros)
        h, hpk = _update(parts, h, g1[l].reshape(1, H), b1[l].reshape(1, H))
    return _head(h, batch.reshape(1, N), Wh, bh.reshape(1, H),
                 gh.reshape(1, H), bh2.reshape(1, H), Wo, bo.reshape(1, 1))


# speedup vs baseline: 3.5747x; 1.0041x over previous
import functools

import jax
import jax.numpy as jnp
from jax import lax
from jax.experimental import pallas as pl
from jax.experimental.pallas import tpu as pltpu
from jax.experimental.pallas import tpu_sc as plsc

N = 10000
E = 320000
D = 128
DE = 16
H = 128
G = 64
L = 3

HP = H // 2
EH = E // 2

NC = 2
NS = 16
NW = NC * NS
EPW = E // NW
KCH = 80
NCHUNK = EPW // KCH
GB = 400
GSUB = 80
NGB = EPW // GB
NSUB = GB // GSUB
SB = 200
SSUB = 40
NSB = EPW // SB
NSS = SB // SSUB
NP = 10240
ROWS_PER_TILE = NP // NS

_mesh = plsc.VectorSubcoreMesh(core_axis_name="c", subcore_axis_name="s")


def _sc_gather_body(hpk_hbm, dst_hbm, src_hbm, hd_hbm, hs_hbm,
                    didx, sidx, dbuf, sbuf, sem_d, sem_s):
    wid = lax.axis_index("s") * NC + lax.axis_index("c")
    base = wid * EPW

    def make_blk(lane, rbase):
        def blk(i, carry):
            off = base + i * GB
            row = rbase + i * GB
            pltpu.sync_copy(dst_hbm.at[pl.ds(off, GB)], didx)
            pltpu.sync_copy(src_hbm.at[pl.ds(off, GB)], sidx)
            copies = []
            for j in range(NSUB):
                sl = pl.ds(j * GSUB, GSUB)
                copies.append(pltpu.async_copy(hpk_hbm.at[didx.at[sl]],
                                               dbuf.at[sl], sem_d))
                copies.append(pltpu.async_copy(hpk_hbm.at[sidx.at[sl]],
                                               sbuf.at[sl], sem_s))
            for c in copies:
                c.wait()
            pltpu.sync_copy(dbuf, hd_hbm.at[pl.ds(row, GB), pl.ds(lane, HP)])
            pltpu.sync_copy(sbuf, hs_hbm.at[pl.ds(row, GB), pl.ds(lane, HP)])
            return carry
        return blk

    @pl.when(wid < NS)
    def _():
        lax.fori_loop(0, NGB, make_blk(0, wid * EPW), 0)

    @pl.when(wid >= NS)
    def _():
        lax.fori_loop(0, NGB, make_blk(HP, (wid - NS) * EPW), 0)


_sc_gather = pl.kernel(
    _sc_gather_body,
    out_type=(jax.ShapeDtypeStruct((EH, H), jnp.uint32),
              jax.ShapeDtypeStruct((EH, H), jnp.uint32)),
    mesh=_mesh,
    compiler_params=pltpu.CompilerParams(use_tc_tiling_on_sc=False),
    scratch_types=[
        pltpu.VMEM((GB,), jnp.int32),
        pltpu.VMEM((GB,), jnp.int32),
        pltpu.VMEM((GB, HP), jnp.uint32),
        pltpu.VMEM((GB, HP), jnp.uint32),
        pltpu.SemaphoreType.DMA,
        pltpu.SemaphoreType.DMA,
    ],
)


def _sc_scatter_body(msgA_hbm, msgB_hbm, dst_hbm, zeros_hbm, out_hbm,
                     didx2, mbuf, acc, sem):
    cid = lax.axis_index("c")
    sid = lax.axis_index("s")
    wid = sid * NC + cid
    base = wid * EPW

    pltpu.sync_copy(zeros_hbm.at[pl.ds(sid * ROWS_PER_TILE, ROWS_PER_TILE)],
                    acc.at[pl.ds(sid * ROWS_PER_TILE, ROWS_PER_TILE)])
    plsc.subcore_barrier()

    def make_blk(msg_hbm, rbase):
        def blk(i, carry):
            off = base + i * SB
            row = rbase + i * SB
            for j in range(NSS):
                pltpu.sync_copy(dst_hbm.at[pl.ds(off + j * SSUB, SSUB)],
                                didx2.at[j])
            pltpu.sync_copy(msg_hbm.at[pl.ds(row, SB)], mbuf)
            copies = []
            for j in range(NSS):
                copies.append(pltpu.async_copy(
                    mbuf.at[pl.ds(j * SSUB, SSUB)], acc.at[didx2.at[j]],
                    sem, add=True))
            for c in copies:
                c.wait()
            return carry
        return blk

    @pl.when(wid < NS)
    def _():
        lax.fori_loop(0, NSB, make_blk(msgA_hbm, wid * EPW), 0)

    @pl.when(wid >= NS)
    def _():
        lax.fori_loop(0, NSB, make_blk(msgB_hbm, (wid - NS) * EPW), 0)

    plsc.subcore_barrier()
    pltpu.sync_copy(acc.at[pl.ds(sid * ROWS_PER_TILE, ROWS_PER_TILE)],
                    out_hbm.at[pl.ds(cid * NP + sid * ROWS_PER_TILE,
                                     ROWS_PER_TILE)])


_sc_scatter = pl.kernel(
    _sc_scatter_body,
    out_type=jax.ShapeDtypeStruct((2 * NP, H), jnp.float32),
    mesh=_mesh,
    scratch_types=[
        pltpu.VMEM((NSS, SSUB), jnp.int32),
        pltpu.VMEM((SB, H), jnp.float32),
        pltpu.VMEM_SHARED((NP, H), jnp.float32),
        pltpu.SemaphoreType.DMA,
    ],
)


def _pack(x):
    u = lax.bitcast_convert_type(x, jnp.uint32)
    r = (u + jnp.uint32(0x7FFF) + ((u >> 16) & jnp.uint32(1))) >> 16
    return r[:, :HP] | (r[:, HP:] << 16)


def _unpack(pk):
    lo = lax.bitcast_convert_type(pk << 16, jnp.float32)
    hi = lax.bitcast_convert_type(pk & jnp.uint32(0xFFFF0000), jnp.float32)
    return jnp.concatenate([lo, hi], axis=1)


def _embed_body(x_ref, w_ref, b_ref, o_ref, opk_ref):
    h0 = (jnp.dot(x_ref[...], w_ref[...],
                  preferred_element_type=jnp.float32) + b_ref[...])
    o_ref[...] = h0
    opk_ref[...] = _pack(h0)


_embed = pl.pallas_call(
    _embed_body,
    out_shape=(jax.ShapeDtypeStruct((N, H), jnp.float32),
               jax.ShapeDtypeStruct((N, HP), jnp.uint32)),
)

BEH = 2000


def _eembed_body(eaA_ref, eaB_ref, w_ref, b_ref, o_ref):
    eA = (jnp.dot(eaA_ref[...], w_ref[...],
                  preferred_element_type=jnp.float32) + b_ref[...])
    eB = (jnp.dot(eaB_ref[...], w_ref[...],
                  preferred_element_type=jnp.float32) + b_ref[...])
    o_ref[...] = jnp.concatenate([_pack(eA), _pack(eB)], axis=1)


_eembed = pl.pallas_call(
    _eembed_body,
    grid=(EH // BEH,),
    in_specs=[
        pl.BlockSpec((BEH, DE), lambda i: (i, 0)),
        pl.BlockSpec((BEH, DE), lambda i: (i + EH // BEH, 0)),
        pl.BlockSpec((DE, H), lambda i: (0, 0)),
        pl.BlockSpec((1, H), lambda i: (0, 0)),
    ],
    out_specs=pl.BlockSpec((BEH, H), lambda i: (i, 0)),
    out_shape=jax.ShapeDtypeStruct((EH, H), jnp.uint32),
)


def _msg_body(hd_ref, hs_ref, e_ref, w_ref, b_ref, oA_ref, oB_ref):
    hdp = hd_ref[...]
    hsp = hs_ref[...]
    ep = e_ref[...]
    zA = jnp.concatenate([_unpack(hdp[:, :HP]), _unpack(hsp[:, :HP]),
                          _unpack(ep[:, :HP])], axis=1)
    zB = jnp.concatenate([_unpack(hdp[:, HP:]), _unpack(hsp[:, HP:]),
                          _unpack(ep[:, HP:])], axis=1)
    z = jnp.concatenate([zA, zB], axis=0)
    logits = (jnp.dot(z, w_ref[...], preferred_element_type=jnp.float32)
              + b_ref[...])
    msg = jax.nn.sigmoid(logits[:, :H]) * jax.nn.softplus(logits[:, H:])
    oA_ref[...] = msg[:BEH]
    oB_ref[...] = msg[BEH:]


_msg = pl.pallas_call(
    _msg_body,
    grid=(EH // BEH,),
    in_specs=[
        pl.BlockSpec((BEH, H), lambda i: (i, 0)),
        pl.BlockSpec((BEH, H), lambda i: (i, 0)),
        pl.BlockSpec((BEH, H), lambda i: (i, 0)),
        pl.BlockSpec((3 * H, 2 * H), lambda i: (0, 0)),
        pl.BlockSpec((1, 2 * H), lambda i: (0, 0)),
    ],
    out_specs=(pl.BlockSpec((BEH, H), lambda i: (i, 0)),
               pl.BlockSpec((BEH, H), lambda i: (i, 0))),
    out_shape=(jax.ShapeDtypeStruct((EH, H), jnp.float32),
               jax.ShapeDtypeStruct((EH, H), jnp.float32)),
)


def _update_body(p_ref, h_ref, g_ref, b_ref, o_ref, opk_ref):
    agg = p_ref[:N, :] + p_ref[NP:NP + N, :]
    mu = jnp.mean(agg, axis=0, keepdims=True)
    xc = agg - mu
    var = jnp.mean(xc * xc, axis=0, keepdims=True)
    xn = xc / jnp.sqrt(var + 1e-5) * g_ref[...] + b_ref[...]
    h_new = h_ref[...] + xn * jax.nn.sigmoid(xn)
    o_ref[...] = h_new
    opk_ref[...] = _pack(h_new)


_update = pl.pallas_call(
    _update_body,
    out_shape=(jax.ShapeDtypeStruct((N, H), jnp.float32),
               jax.ShapeDtypeStruct((N, HP), jnp.uint32)),
)


def _head_body(h_ref, batch_ref, wh_ref, bh_ref, gh_ref, bh2_ref, wo_ref,
               bo_ref, o_ref):
    ids = lax.broadcasted_iota(jnp.int32, (G, N), 0)
    onehot = jnp.where(ids == batch_ref[...], 1.0, 0.0).astype(jnp.float32)
    pooled = jnp.dot(onehot, h_ref[...], preferred_element_type=jnp.float32,
                     precision=lax.Precision.HIGHEST)
    t0 = (jnp.dot(pooled, wh_ref[...], preferred_element_type=jnp.float32)
          + bh_ref[...])
    mu = jnp.mean(t0, axis=0, keepdims=True)
    xc = t0 - mu
    var = jnp.mean(xc * xc, axis=0, keepdims=True)
    xn = xc / jnp.sqrt(var + 1e-5) * gh_ref[...] + bh2_ref[...]
    t = xn * jax.nn.sigmoid(xn)
    o_ref[...] = (jnp.dot(t, wo_ref[...], preferred_element_type=jnp.float32)
                  + bo_ref[...])


_head = pl.pallas_call(
    _head_body,
    out_shape=jax.ShapeDtypeStruct((G, 1), jnp.float32),
)


def kernel(x, edge_index, edge_attr, batch, Wn, bn_b, We, be, Wf, bf, Ws, bs,
           g1, b1, Wh, bh, gh, bh2, Wo, bo):
    src = edge_index[0]
    dst = edge_index[1]
    h, hpk = _embed(x, Wn, bn_b.reshape(1, H))
    epk = _eembed(edge_attr, edge_attr, We, be.reshape(1, H))
    zeros = jnp.zeros((NP, H), jnp.float32)
    for l in range(L):
        w = jnp.concatenate([Wf[l], Ws[l]], axis=1)
        b2 = jnp.concatenate([bf[l], bs[l]]).reshape(1, 2 * H)
        hd2, hs2 = _sc_gather(hpk, dst, src)
        msgA, msgB = _msg(hd2, hs2, epk, w, b2)
        parts = _sc_scatter(msgA, msgB, dst, zeros)
        h, hpk = _update(parts, h, g1[l].reshape(1, H), b1[l].reshape(1, H))
    return _head(h, batch.reshape(1, N), Wh, bh.reshape(1, H),
                 gh.reshape(1, H), bh2.reshape(1, H), Wo, bo.reshape(1, 1))
